# Initial kernel scaffold; baseline (speedup 1.0000x reference)
import jax
import jax.numpy as jnp
from jax.experimental import pallas as pl


def kernel(x, W1, b1, W2, b2, W3, b3, Wb, bb, Wf1, bf1, Wf2, bf2, gamma, beta):
    raise NotImplementedError("write your pallas kernel here")


# speedup vs baseline: 13.9075x; 13.9075x over previous
import functools

import jax
import jax.numpy as jnp
from jax.experimental import pallas as pl
from jax.experimental.pallas import tpu as pltpu

H = 16
S = 2048
D = 1024
HD = 64

BQ = 256
NQ = S // BQ
BV = 512
BM = 256

_SCALE = 1.0 / (D ** 0.5)


def _gelu(x):
    return x * 0.5 * (1.0 + jax.lax.erf(x * (2.0 ** -0.5)))


def _qkv_body(x_ref, w_ref, b_ref, o_ref):
    x = x_ref[...].astype(jnp.bfloat16)
    w = w_ref[0].astype(jnp.bfloat16)
    acc = jax.lax.dot_general(x, w, (((1,), (1,)), ((), ())),
                              preferred_element_type=jnp.float32)
    o_ref[0] = acc + b_ref[0]


def _attn_body(q_ref, k_ref, v_ref, o_ref, gexp_ref, vg_ref, idx_ref):
    hh = pl.program_id(0)
    for j in range(2):
        q = q_ref[j].astype(jnp.bfloat16)
        k = k_ref[j].astype(jnp.bfloat16)
        v = v_ref[j].astype(jnp.bfloat16)
        s = jax.lax.dot_general(q, k, (((1,), (1,)), ((), ())),
                                preferred_element_type=jnp.float32) * _SCALE
        m = jnp.max(s, axis=1, keepdims=True)
        iota = jax.lax.broadcasted_iota(jnp.int32, s.shape, 1)
        idx = jnp.min(jnp.where(s == m, iota, S), axis=1)
        p = jnp.exp(s - m)
        l = jnp.sum(p, axis=1, keepdims=True)
        pv = jax.lax.dot_general(p.astype(jnp.bfloat16), v,
                                 (((1,), (0,)), ((), ())),
                                 preferred_element_type=jnp.float32)
        ginv = 1.0 / l
        sl = slice(j * HD, (j + 1) * HD)
        o_ref[:, sl] = pv * ginv
        gexp_ref[:, sl] = jnp.broadcast_to(ginv, (BQ, HD))
        onehot = (iota == idx[:, None]).astype(jnp.bfloat16)
        vg_ref[:, sl] = jax.lax.dot_general(onehot, v,
                                            (((1,), (0,)), ((), ())),
                                            preferred_element_type=jnp.float32)
        idx_ref[0, 0, j] = idx + (2 * hh + j) * S


def _vb_body(g_ref, wb_ref, bb_ref, v_ref):
    g = g_ref[...].astype(jnp.bfloat16)
    wb = wb_ref[...].astype(jnp.bfloat16)
    acc = jax.lax.dot_general(g, wb, (((1,), (1,)), ((), ())),
                              preferred_element_type=jnp.float32)
    v_ref[...] = _gelu(acc + bb_ref[...])


def _ffn_body(o_ref, gexp_ref, vg_ref, v_ref, wf1_ref, bf1_ref,
              wf2_ref, bf2_ref, gam_ref, bet_ref, y_ref):
    vcol = v_ref[...]
    x0 = o_ref[...] + (vcol - gexp_ref[...]) * vg_ref[...]
    h1 = jax.lax.dot_general(x0.astype(jnp.bfloat16),
                             wf1_ref[...].astype(jnp.bfloat16),
                             (((1,), (1,)), ((), ())),
                             preferred_element_type=jnp.float32)
    h1 = _gelu(h1 + bf1_ref[...])
    h2 = jax.lax.dot_general(h1.astype(jnp.bfloat16),
                             wf2_ref[...].astype(jnp.bfloat16),
                             (((1,), (1,)), ((), ())),
                             preferred_element_type=jnp.float32)
    h2 = h2 + bf2_ref[...]
    mu = jnp.mean(h2, axis=1, keepdims=True)
    cen = h2 - mu
    var = jnp.mean(cen * cen, axis=1, keepdims=True)
    y_ref[...] = cen * jax.lax.rsqrt(var + 1e-5) * gam_ref[...] + bet_ref[...]


def kernel(x, W1, b1, W2, b2, W3, b3, Wb, bb, Wf1, bf1, Wf2, bf2,
           gamma, beta):
    xs = x.reshape(S, D)
    w_all = jnp.concatenate([W1, W2, W3], axis=0).reshape(3 * H, HD, D)
    b_all = jnp.concatenate([b1, b2, b3], axis=0).reshape(3 * H, 1, HD)

    qkv = pl.pallas_call(
        _qkv_body,
        grid=(3 * H,),
        in_specs=[
            pl.BlockSpec((S, D), lambda j: (0, 0)),
            pl.BlockSpec((1, HD, D), lambda j: (j, 0, 0)),
            pl.BlockSpec((1, 1, HD), lambda j: (j, 0, 0)),
        ],
        out_specs=pl.BlockSpec((1, S, HD), lambda j: (j, 0, 0)),
        out_shape=jax.ShapeDtypeStruct((3 * H, S, HD), jnp.float32),
    )(xs, w_all, b_all)

    Q, K, V = qkv[:H], qkv[H:2 * H], qkv[2 * H:]

    O, Gexp, Vg, idxg = pl.pallas_call(
        _attn_body,
        grid=(H // 2, NQ),
        in_specs=[
            pl.BlockSpec((2, BQ, HD), lambda hh, q: (hh, q, 0)),
            pl.BlockSpec((2, S, HD), lambda hh, q: (hh, 0, 0)),
            pl.BlockSpec((2, S, HD), lambda hh, q: (hh, 0, 0)),
        ],
        out_specs=[
            pl.BlockSpec((BQ, 2 * HD), lambda hh, q: (q, hh)),
            pl.BlockSpec((BQ, 2 * HD), lambda hh, q: (q, hh)),
            pl.BlockSpec((BQ, 2 * HD), lambda hh, q: (q, hh)),
            pl.BlockSpec((1, 1, 2, BQ), lambda hh, q: (hh, q, 0, 0)),
        ],
        out_shape=[
            jax.ShapeDtypeStruct((S, D), jnp.float32),
            jax.ShapeDtypeStruct((S, D), jnp.float32),
            jax.ShapeDtypeStruct((S, D), jnp.float32),
            jax.ShapeDtypeStruct((H // 2, NQ, 2, BQ), jnp.int32),
        ],
    )(Q, K, V)
    del idxg

    g_row = jax.lax.slice(Gexp, (0, (H - 1) * HD), (S, (H - 1) * HD + 1))
    g_row = g_row.reshape(1, S)

    v_vec = pl.pallas_call(
        _vb_body,
        grid=(S // BV,),
        in_specs=[
            pl.BlockSpec((1, S), lambda i: (0, 0)),
            pl.BlockSpec((BV, S), lambda i: (i, 0)),
            pl.BlockSpec((1, BV), lambda i: (0, i)),
        ],
        out_specs=pl.BlockSpec((1, BV), lambda i: (0, i)),
        out_shape=jax.ShapeDtypeStruct((1, S), jnp.float32),
    )(g_row, Wb, bb.reshape(1, S))
    v_col = v_vec.reshape(S, 1)

    y = pl.pallas_call(
        _ffn_body,
        grid=(S // BM,),
        in_specs=[
            pl.BlockSpec((BM, D), lambda i: (i, 0)),
            pl.BlockSpec((BM, D), lambda i: (i, 0)),
            pl.BlockSpec((BM, D), lambda i: (i, 0)),
            pl.BlockSpec((BM, 1), lambda i: (i, 0)),
            pl.BlockSpec((4 * D, D), lambda i: (0, 0)),
            pl.BlockSpec((1, 4 * D), lambda i: (0, 0)),
            pl.BlockSpec((D, 4 * D), lambda i: (0, 0)),
            pl.BlockSpec((1, D), lambda i: (0, 0)),
            pl.BlockSpec((1, D), lambda i: (0, 0)),
            pl.BlockSpec((1, D), lambda i: (0, 0)),
        ],
        out_specs=pl.BlockSpec((BM, D), lambda i: (i, 0)),
        out_shape=jax.ShapeDtypeStruct((S, D), jnp.float32),
    )(O, Gexp, Vg, v_col, Wf1, bf1.reshape(1, 4 * D), Wf2,
      bf2.reshape(1, D), gamma.reshape(1, D), beta.reshape(1, D))

    return y.reshape(1, S, D)
